# Initial kernel scaffold; baseline (speedup 1.0000x reference)
import jax
import jax.numpy as jnp
from jax.experimental import pallas as pl


def kernel(ref, query, e_ref, e_query, half_voxel_size):
    raise NotImplementedError("write your pallas kernel here")


# speedup vs baseline: 18.1481x; 18.1481x over previous
import jax
import jax.numpy as jnp
from jax import lax
from jax.experimental import pallas as pl
from jax.experimental.pallas import tpu as pltpu
from jax.experimental.pallas import tpu_sc as plsc

NC = 2
NS = 16
NW = NC * NS
LANES = 16
ROWW = 16
BLK = 128
K = 10
CH = K * BLK


def _body(ref_h, query_h, eref_h, equery_h, hvec_h, out_h,
          idx_r, idx_q, rows_r, rows_q, out_v, hv_v, sem):
  n_chunks = out_h.shape[0] // CH

  wid = lax.axis_index("s") * NC + lax.axis_index("c")
  base_cnt = n_chunks // NW
  rem = n_chunks % NW
  count = base_cnt + jnp.where(wid < rem, 1, 0)
  start = wid * base_cnt + jnp.minimum(wid, rem)

  pltpu.sync_copy(hvec_h, hv_v)
  hx = hv_v[0]
  hy = hv_v[1]
  hz = hv_v[2]

  iota = lax.broadcasted_iota(jnp.int32, (LANES,), 0)
  c1 = jnp.full((LANES,), 1, jnp.int32)
  c2 = jnp.full((LANES,), 2, jnp.int32)
  c3 = jnp.full((LANES,), 3, jnp.int32)

  def chunk_body(j, carry):
    c = start + j
    pltpu.sync_copy(eref_h.at[pl.ds(c * CH, CH)], idx_r)
    pltpu.sync_copy(equery_h.at[pl.ds(c * CH, CH)], idx_q)
    copies = []
    for k in range(K):
      copies.append(
          pltpu.async_copy(ref_h.at[idx_r.at[pl.ds(k * BLK, BLK)]],
                           rows_r.at[pl.ds(k * BLK, BLK)], sem))
      copies.append(
          pltpu.async_copy(query_h.at[idx_q.at[pl.ds(k * BLK, BLK)]],
                           rows_q.at[pl.ds(k * BLK, BLK)], sem))
    for cp in copies:
      cp.wait()

    def grp(g, carry2):
      pos = g * LANES + iota
      r1 = plsc.load_gather(rows_r, [pos, c1]) - plsc.load_gather(rows_q, [pos, c1])
      r2 = plsc.load_gather(rows_r, [pos, c2]) - plsc.load_gather(rows_q, [pos, c2])
      r3 = plsc.load_gather(rows_r, [pos, c3]) - plsc.load_gather(rows_q, [pos, c3])
      o1 = (r1 >= hx).astype(jnp.int32) + (r1 > -hx).astype(jnp.int32)
      o2 = (r2 >= hy).astype(jnp.int32) + (r2 > -hy).astype(jnp.int32)
      o3 = (r3 >= hz).astype(jnp.int32) + (r3 > -hz).astype(jnp.int32)
      out_v[pl.ds(g * LANES, LANES)] = (o3 * 3 + o2) * 3 + o1
      return carry2

    lax.fori_loop(0, CH // LANES, grp, 0)
    pltpu.sync_copy(out_v, out_h.at[pl.ds(c * CH, CH)])
    return carry

  lax.fori_loop(0, count, chunk_body, 0)


def kernel(ref, query, e_ref, e_query, half_voxel_size):
  e = e_ref.shape[0]
  assert e % CH == 0, e
  ref16 = jnp.pad(ref, ((0, 0), (0, ROWW - ref.shape[1])))
  query16 = jnp.pad(query, ((0, 0), (0, ROWW - query.shape[1])))
  hvec = jnp.broadcast_to(half_voxel_size.astype(jnp.float32).reshape(3, 1),
                          (3, LANES))

  mesh = plsc.VectorSubcoreMesh(core_axis_name="c", subcore_axis_name="s")
  run = pl.kernel(
      _body,
      out_type=jax.ShapeDtypeStruct((e,), jnp.int32),
      mesh=mesh,
      scratch_types=[
          pltpu.VMEM((CH,), jnp.int32),
          pltpu.VMEM((CH,), jnp.int32),
          pltpu.VMEM((CH, ROWW), jnp.float32),
          pltpu.VMEM((CH, ROWW), jnp.float32),
          pltpu.VMEM((CH,), jnp.int32),
          pltpu.VMEM((3, LANES), jnp.float32),
          pltpu.SemaphoreType.DMA,
      ],
      compiler_params=pltpu.CompilerParams(
          needs_layout_passes=False, use_tc_tiling_on_sc=False),
  )
  return run(ref16, query16, e_ref, e_query, hvec)


# speedup vs baseline: 27.1075x; 1.4937x over previous
import jax
import jax.numpy as jnp
from jax import lax
from jax.experimental import pallas as pl
from jax.experimental.pallas import tpu as pltpu
from jax.experimental.pallas import tpu_sc as plsc

NC = 2
NS = 16
NW = NC * NS
LANES = 16
ROWW = 16
BLK = 128
K = 10
CH = K * BLK


def _body(ref_h, query_h, eref_h, equery_h, hvec_h, out_h,
          idx_r, idx_q, rows_r, rows_q, out_v, hv_v, sem0, sem1):
  n_chunks = out_h.shape[0] // CH

  wid = lax.axis_index("s") * NC + lax.axis_index("c")
  base_cnt = n_chunks // NW
  rem = n_chunks % NW
  count = base_cnt + jnp.where(wid < rem, 1, 0)
  start = wid * base_cnt + jnp.minimum(wid, rem)

  pltpu.sync_copy(hvec_h, hv_v)
  hx = hv_v[0]
  hy = hv_v[1]
  hz = hv_v[2]

  iota = lax.broadcasted_iota(jnp.int32, (LANES,), 0)
  c1 = jnp.full((LANES,), 1, jnp.int32)
  c2 = jnp.full((LANES,), 2, jnp.int32)
  c3 = jnp.full((LANES,), 3, jnp.int32)

  def fire(slot, c, sem):
    pltpu.sync_copy(eref_h.at[pl.ds(c * CH, CH)], idx_r.at[slot])
    pltpu.sync_copy(equery_h.at[pl.ds(c * CH, CH)], idx_q.at[slot])
    for k in range(K):
      pltpu.async_copy(ref_h.at[idx_r.at[slot, pl.ds(k * BLK, BLK)]],
                       rows_r.at[slot, pl.ds(k * BLK, BLK)], sem)
      pltpu.async_copy(query_h.at[idx_q.at[slot, pl.ds(k * BLK, BLK)]],
                       rows_q.at[slot, pl.ds(k * BLK, BLK)], sem)

  def drain(slot, sem):
    pltpu.make_async_copy(ref_h.at[pl.ds(0, CH)], rows_r.at[slot], sem).wait()
    pltpu.make_async_copy(query_h.at[pl.ds(0, CH)], rows_q.at[slot], sem).wait()

  @pl.when(count > 0)
  def _():
    fire(0, start, sem0)

  def chunk_body(j, carry):
    p = lax.rem(j, 2)
    has_next = j + 1 < count

    @pl.when(jnp.logical_and(has_next, p == 0))
    def _():
      fire(1, start + j + 1, sem1)

    @pl.when(jnp.logical_and(has_next, p == 1))
    def _():
      fire(0, start + j + 1, sem0)

    @pl.when(p == 0)
    def _():
      drain(0, sem0)

    @pl.when(p == 1)
    def _():
      drain(1, sem1)

    rr = rows_r.at[p]
    rq = rows_q.at[p]

    def grp(g, carry2):
      pos = g * LANES + iota
      r1 = plsc.load_gather(rr, [pos, c1]) - plsc.load_gather(rq, [pos, c1])
      r2 = plsc.load_gather(rr, [pos, c2]) - plsc.load_gather(rq, [pos, c2])
      r3 = plsc.load_gather(rr, [pos, c3]) - plsc.load_gather(rq, [pos, c3])
      o1 = (r1 >= hx).astype(jnp.int32) + (r1 > -hx).astype(jnp.int32)
      o2 = (r2 >= hy).astype(jnp.int32) + (r2 > -hy).astype(jnp.int32)
      o3 = (r3 >= hz).astype(jnp.int32) + (r3 > -hz).astype(jnp.int32)
      out_v[pl.ds(g * LANES, LANES)] = (o3 * 3 + o2) * 3 + o1
      return carry2

    lax.fori_loop(0, CH // LANES, grp, 0)
    pltpu.sync_copy(out_v, out_h.at[pl.ds((start + j) * CH, CH)])
    return carry

  lax.fori_loop(0, count, chunk_body, 0)


def kernel(ref, query, e_ref, e_query, half_voxel_size):
  e = e_ref.shape[0]
  assert e % CH == 0, e
  ref16 = jnp.pad(ref, ((0, 0), (0, ROWW - ref.shape[1])))
  query16 = jnp.pad(query, ((0, 0), (0, ROWW - query.shape[1])))
  hvec = jnp.broadcast_to(half_voxel_size.astype(jnp.float32).reshape(3, 1),
                          (3, LANES))

  mesh = plsc.VectorSubcoreMesh(core_axis_name="c", subcore_axis_name="s")
  run = pl.kernel(
      _body,
      out_type=jax.ShapeDtypeStruct((e,), jnp.int32),
      mesh=mesh,
      scratch_types=[
          pltpu.VMEM((2, CH), jnp.int32),
          pltpu.VMEM((2, CH), jnp.int32),
          pltpu.VMEM((2, CH, ROWW), jnp.float32),
          pltpu.VMEM((2, CH, ROWW), jnp.float32),
          pltpu.VMEM((CH,), jnp.int32),
          pltpu.VMEM((3, LANES), jnp.float32),
          pltpu.SemaphoreType.DMA,
          pltpu.SemaphoreType.DMA,
      ],
      compiler_params=pltpu.CompilerParams(
          needs_layout_passes=False, use_tc_tiling_on_sc=False),
  )
  return run(ref16, query16, e_ref, e_query, hvec)


# speedup vs baseline: 29.6920x; 1.0953x over previous
import jax
import jax.numpy as jnp
from jax import lax
from jax.experimental import pallas as pl
from jax.experimental.pallas import tpu as pltpu
from jax.experimental.pallas import tpu_sc as plsc

NC = 2
NS = 16
NW = NC * NS
LANES = 16
ROWW = 16
BLK = 128
K = 10
CH = K * BLK


def _body(ref_h, query_h, eref_h, equery_h, hvec_h, out_h,
          idx_r, idx_q, rows_r, rows_q, out_v, hv_v,
          sem0, sem1, semi, semo0, semo1):
  n_chunks = out_h.shape[0] // CH

  wid = lax.axis_index("s") * NC + lax.axis_index("c")
  base_cnt = n_chunks // NW
  rem = n_chunks % NW
  count = base_cnt + jnp.where(wid < rem, 1, 0)
  start = wid * base_cnt + jnp.minimum(wid, rem)

  pltpu.sync_copy(hvec_h, hv_v)
  hx = hv_v[0]
  hy = hv_v[1]
  hz = hv_v[2]

  iota = lax.broadcasted_iota(jnp.int32, (LANES,), 0)
  c1 = jnp.full((LANES,), 1, jnp.int32)
  c2 = jnp.full((LANES,), 2, jnp.int32)
  c3 = jnp.full((LANES,), 3, jnp.int32)

  def stage_idx(slot, c):
    pltpu.async_copy(eref_h.at[pl.ds(c * CH, CH)], idx_r.at[slot], semi)
    pltpu.async_copy(equery_h.at[pl.ds(c * CH, CH)], idx_q.at[slot], semi)

  def drain_idx(slot):
    pltpu.make_async_copy(eref_h.at[pl.ds(0, CH)], idx_r.at[slot], semi).wait()
    pltpu.make_async_copy(equery_h.at[pl.ds(0, CH)], idx_q.at[slot], semi).wait()

  def fire(slot, sem):
    for k in range(K):
      pltpu.async_copy(ref_h.at[idx_r.at[slot, pl.ds(k * BLK, BLK)]],
                       rows_r.at[slot, pl.ds(k * BLK, BLK)], sem)
      pltpu.async_copy(query_h.at[idx_q.at[slot, pl.ds(k * BLK, BLK)]],
                       rows_q.at[slot, pl.ds(k * BLK, BLK)], sem)

  def drain(slot, sem):
    pltpu.make_async_copy(ref_h.at[pl.ds(0, CH)], rows_r.at[slot], sem).wait()
    pltpu.make_async_copy(query_h.at[pl.ds(0, CH)], rows_q.at[slot], sem).wait()

  def drain_out(slot, sem):
    pltpu.make_async_copy(out_v.at[slot], out_h.at[pl.ds(0, CH)], sem).wait()

  pltpu.sync_copy(eref_h.at[pl.ds(start * CH, CH)], idx_r.at[0])
  pltpu.sync_copy(equery_h.at[pl.ds(start * CH, CH)], idx_q.at[0])
  fire(0, sem0)
  stage_idx(1, start + 1)

  def chunk_body(j, carry):
    p = lax.rem(j, 2)
    has_next = j + 1 < count

    @pl.when(jnp.logical_and(has_next, p == 0))
    def _():
      drain_idx(1)
      fire(1, sem1)

    @pl.when(jnp.logical_and(has_next, p == 1))
    def _():
      drain_idx(0)
      fire(0, sem0)

    @pl.when(p == 0)
    def _():
      drain(0, sem0)

    @pl.when(p == 1)
    def _():
      drain(1, sem1)

    @pl.when(j + 2 < count)
    def _():
      stage_idx(p, start + j + 2)

    @pl.when(jnp.logical_and(j >= 2, p == 0))
    def _():
      drain_out(0, semo0)

    @pl.when(jnp.logical_and(j >= 2, p == 1))
    def _():
      drain_out(1, semo1)

    rr = rows_r.at[p]
    rq = rows_q.at[p]
    ov = out_v.at[p]

    def grp(g, carry2):
      pos = g * LANES + iota
      r1 = plsc.load_gather(rr, [pos, c1]) - plsc.load_gather(rq, [pos, c1])
      r2 = plsc.load_gather(rr, [pos, c2]) - plsc.load_gather(rq, [pos, c2])
      r3 = plsc.load_gather(rr, [pos, c3]) - plsc.load_gather(rq, [pos, c3])
      o1 = (r1 >= hx).astype(jnp.int32) + (r1 > -hx).astype(jnp.int32)
      o2 = (r2 >= hy).astype(jnp.int32) + (r2 > -hy).astype(jnp.int32)
      o3 = (r3 >= hz).astype(jnp.int32) + (r3 > -hz).astype(jnp.int32)
      ov[pl.ds(g * LANES, LANES)] = (o3 * 3 + o2) * 3 + o1
      return carry2

    lax.fori_loop(0, CH // LANES, grp, 0)

    @pl.when(p == 0)
    def _():
      pltpu.async_copy(out_v.at[0], out_h.at[pl.ds((start + j) * CH, CH)], semo0)

    @pl.when(p == 1)
    def _():
      pltpu.async_copy(out_v.at[1], out_h.at[pl.ds((start + j) * CH, CH)], semo1)

    return carry

  lax.fori_loop(0, count, chunk_body, 0)
  drain_out(0, semo0)
  drain_out(1, semo1)


def kernel(ref, query, e_ref, e_query, half_voxel_size):
  e = e_ref.shape[0]
  assert e % CH == 0, e
  assert (e // CH) // NW >= 2, e
  ref16 = jnp.pad(ref, ((0, 0), (0, ROWW - ref.shape[1])))
  query16 = jnp.pad(query, ((0, 0), (0, ROWW - query.shape[1])))
  hvec = jnp.broadcast_to(half_voxel_size.astype(jnp.float32).reshape(3, 1),
                          (3, LANES))

  mesh = plsc.VectorSubcoreMesh(core_axis_name="c", subcore_axis_name="s")
  run = pl.kernel(
      _body,
      out_type=jax.ShapeDtypeStruct((e,), jnp.int32),
      mesh=mesh,
      scratch_types=[
          pltpu.VMEM((2, CH), jnp.int32),
          pltpu.VMEM((2, CH), jnp.int32),
          pltpu.VMEM((2, CH, ROWW), jnp.float32),
          pltpu.VMEM((2, CH, ROWW), jnp.float32),
          pltpu.VMEM((2, CH), jnp.int32),
          pltpu.VMEM((3, LANES), jnp.float32),
          pltpu.SemaphoreType.DMA,
          pltpu.SemaphoreType.DMA,
          pltpu.SemaphoreType.DMA,
          pltpu.SemaphoreType.DMA,
          pltpu.SemaphoreType.DMA,
      ],
      compiler_params=pltpu.CompilerParams(
          needs_layout_passes=False, use_tc_tiling_on_sc=False),
  )
  return run(ref16, query16, e_ref, e_query, hvec)


# speedup vs baseline: 31.8316x; 1.0721x over previous
import jax
import jax.numpy as jnp
from jax import lax
from jax.experimental import pallas as pl
from jax.experimental.pallas import tpu as pltpu
from jax.experimental.pallas import tpu_sc as plsc

NC = 2
NS = 16
NW = NC * NS
LANES = 16
ROWW = 16
BLK = 128
K = 10
CH = K * BLK


def _body(ref_h, query_h, eref_h, equery_h, hvec_h, out_h,
          idx_r, idx_q, idxh_r, idxh_q, rows_r, rows_q, out_v, hv_v,
          sem0, sem1, semi, semo0, semo1):
  n_chunks = out_h.shape[0] // CH

  wid = lax.axis_index("s") * NC + lax.axis_index("c")
  base_cnt = n_chunks // NW
  rem = n_chunks % NW
  count = base_cnt + jnp.where(wid < rem, 1, 0)
  start = wid * base_cnt + jnp.minimum(wid, rem)

  pltpu.sync_copy(hvec_h, hv_v)
  hx = hv_v[0]
  hy = hv_v[1]
  hz = hv_v[2]

  iota = lax.broadcasted_iota(jnp.int32, (LANES,), 0)

  def stage_idx(slot, c):
    pltpu.async_copy(eref_h.at[pl.ds(c * CH, CH)], idx_r.at[slot], semi)
    pltpu.async_copy(equery_h.at[pl.ds(c * CH, CH)], idx_q.at[slot], semi)

  def drain_idx(slot):
    pltpu.make_async_copy(eref_h.at[pl.ds(0, CH)], idx_r.at[slot], semi).wait()
    pltpu.make_async_copy(equery_h.at[pl.ds(0, CH)], idx_q.at[slot], semi).wait()

  def transform_idx(slot):
    def tb(g, carry):
      for u in range(8):
        sl = pl.ds((g * 8 + u) * LANES, LANES)
        idxh_r[slot, sl] = lax.shift_right_logical(idx_r[slot, sl], 2)
        idxh_q[slot, sl] = lax.shift_right_logical(idx_q[slot, sl], 2)
      return carry

    lax.fori_loop(0, CH // (8 * LANES), tb, 0)

  def fire(slot, sem):
    for k in range(K):
      pltpu.async_copy(ref_h.at[idxh_r.at[slot, pl.ds(k * BLK, BLK)]],
                       rows_r.at[slot, pl.ds(k * BLK, BLK)], sem)
      pltpu.async_copy(query_h.at[idxh_q.at[slot, pl.ds(k * BLK, BLK)]],
                       rows_q.at[slot, pl.ds(k * BLK, BLK)], sem)

  def drain(slot, sem):
    pltpu.make_async_copy(ref_h.at[pl.ds(0, CH)], rows_r.at[slot], sem).wait()
    pltpu.make_async_copy(query_h.at[pl.ds(0, CH)], rows_q.at[slot], sem).wait()

  def drain_out(slot, sem):
    pltpu.make_async_copy(out_v.at[slot], out_h.at[pl.ds(0, CH)], sem).wait()

  pltpu.sync_copy(eref_h.at[pl.ds(start * CH, CH)], idx_r.at[0])
  pltpu.sync_copy(equery_h.at[pl.ds(start * CH, CH)], idx_q.at[0])
  transform_idx(0)
  fire(0, sem0)
  stage_idx(1, start + 1)

  def chunk_body(j, carry):
    p = lax.rem(j, 2)
    has_next = j + 1 < count

    @pl.when(jnp.logical_and(has_next, p == 0))
    def _():
      drain_idx(1)
      transform_idx(1)
      fire(1, sem1)

    @pl.when(jnp.logical_and(has_next, p == 1))
    def _():
      drain_idx(0)
      transform_idx(0)
      fire(0, sem0)

    @pl.when(p == 0)
    def _():
      drain(0, sem0)

    @pl.when(p == 1)
    def _():
      drain(1, sem1)

    @pl.when(jnp.logical_and(j >= 2, p == 0))
    def _():
      drain_out(0, semo0)

    @pl.when(jnp.logical_and(j >= 2, p == 1))
    def _():
      drain_out(1, semo1)

    rr = rows_r.at[p]
    rq = rows_q.at[p]
    ov = out_v.at[p]

    def grp(g, carry2):
      sl = pl.ds(g * LANES, LANES)
      pos = g * LANES + iota
      cbr = lax.shift_left(idx_r[p, sl] & 3, 2)
      cbq = lax.shift_left(idx_q[p, sl] & 3, 2)
      r1 = plsc.load_gather(rr, [pos, cbr + 1]) - plsc.load_gather(rq, [pos, cbq + 1])
      r2 = plsc.load_gather(rr, [pos, cbr + 2]) - plsc.load_gather(rq, [pos, cbq + 2])
      r3 = plsc.load_gather(rr, [pos, cbr + 3]) - plsc.load_gather(rq, [pos, cbq + 3])
      o1 = (r1 >= hx).astype(jnp.int32) + (r1 > -hx).astype(jnp.int32)
      o2 = (r2 >= hy).astype(jnp.int32) + (r2 > -hy).astype(jnp.int32)
      o3 = (r3 >= hz).astype(jnp.int32) + (r3 > -hz).astype(jnp.int32)
      ov[sl] = (o3 * 3 + o2) * 3 + o1
      return carry2

    lax.fori_loop(0, CH // LANES, grp, 0)

    @pl.when(j + 2 < count)
    def _():
      stage_idx(p, start + j + 2)

    @pl.when(p == 0)
    def _():
      pltpu.async_copy(out_v.at[0], out_h.at[pl.ds((start + j) * CH, CH)], semo0)

    @pl.when(p == 1)
    def _():
      pltpu.async_copy(out_v.at[1], out_h.at[pl.ds((start + j) * CH, CH)], semo1)

    return carry

  lax.fori_loop(0, count, chunk_body, 0)
  drain_out(0, semo0)
  drain_out(1, semo1)


def kernel(ref, query, e_ref, e_query, half_voxel_size):
  e = e_ref.shape[0]
  assert e % CH == 0, e
  assert (e // CH) // NW >= 2, e
  n, w = ref.shape
  assert w == 4 and ref.shape == query.shape and n % 4 == 0
  ref16 = ref.reshape(n // 4, ROWW)
  query16 = query.reshape(n // 4, ROWW)
  hvec = jnp.broadcast_to(half_voxel_size.astype(jnp.float32).reshape(3, 1),
                          (3, LANES))

  mesh = plsc.VectorSubcoreMesh(core_axis_name="c", subcore_axis_name="s")
  run = pl.kernel(
      _body,
      out_type=jax.ShapeDtypeStruct((e,), jnp.int32),
      mesh=mesh,
      scratch_types=[
          pltpu.VMEM((2, CH), jnp.int32),
          pltpu.VMEM((2, CH), jnp.int32),
          pltpu.VMEM((2, CH), jnp.int32),
          pltpu.VMEM((2, CH), jnp.int32),
          pltpu.VMEM((2, CH, ROWW), jnp.float32),
          pltpu.VMEM((2, CH, ROWW), jnp.float32),
          pltpu.VMEM((2, CH), jnp.int32),
          pltpu.VMEM((3, LANES), jnp.float32),
          pltpu.SemaphoreType.DMA,
          pltpu.SemaphoreType.DMA,
          pltpu.SemaphoreType.DMA,
          pltpu.SemaphoreType.DMA,
          pltpu.SemaphoreType.DMA,
      ],
      compiler_params=pltpu.CompilerParams(
          needs_layout_passes=False, use_tc_tiling_on_sc=False),
  )
  return run(ref16, query16, e_ref, e_query, hvec)


# speedup vs baseline: 32.5393x; 1.0222x over previous
import jax
import jax.numpy as jnp
from jax import lax
from jax.experimental import pallas as pl
from jax.experimental.pallas import tpu as pltpu
from jax.experimental.pallas import tpu_sc as plsc

NC = 2
NS = 16
NW = NC * NS
LANES = 16
ROWW = 16
BLK = 128
K = 10
CH = K * BLK


def _body(ref_h, query_h, eref_h, equery_h, hvec_h, out_h,
          idx_r, idx_q, idxh_r, idxh_q, rows_r, rows_q, out_v, hv_v,
          sem0, sem1, semi, semo0, semo1):
  n_chunks = out_h.shape[0] // CH

  wid = lax.axis_index("s") * NC + lax.axis_index("c")
  base_cnt = n_chunks // NW
  rem = n_chunks % NW
  count = base_cnt + jnp.where(wid < rem, 1, 0)
  start = wid * base_cnt + jnp.minimum(wid, rem)

  pltpu.sync_copy(hvec_h, hv_v)
  hx = hv_v[0]
  hy = hv_v[1]
  hz = hv_v[2]

  iota = lax.broadcasted_iota(jnp.int32, (LANES,), 0)

  def stage_idx(slot, c):
    pltpu.async_copy(eref_h.at[pl.ds(c * CH, CH)], idx_r.at[slot], semi)
    pltpu.async_copy(equery_h.at[pl.ds(c * CH, CH)], idx_q.at[slot], semi)

  def drain_idx(slot):
    pltpu.make_async_copy(eref_h.at[pl.ds(0, CH)], idx_r.at[slot], semi).wait()
    pltpu.make_async_copy(equery_h.at[pl.ds(0, CH)], idx_q.at[slot], semi).wait()

  def transform_idx(slot):
    @plsc.parallel_loop(0, CH // LANES, 1, unroll=8)
    def _(g):
      sl = pl.ds(g * LANES, LANES)
      idxh_r[slot, sl] = lax.shift_right_logical(idx_r[slot, sl], 2)
      idxh_q[slot, sl] = lax.shift_right_logical(idx_q[slot, sl], 2)

  def fire(slot, sem):
    for k in range(K):
      pltpu.async_copy(ref_h.at[idxh_r.at[slot, pl.ds(k * BLK, BLK)]],
                       rows_r.at[slot, pl.ds(k * BLK, BLK)], sem)
      pltpu.async_copy(query_h.at[idxh_q.at[slot, pl.ds(k * BLK, BLK)]],
                       rows_q.at[slot, pl.ds(k * BLK, BLK)], sem)

  def drain(slot, sem):
    pltpu.make_async_copy(ref_h.at[pl.ds(0, CH)], rows_r.at[slot], sem).wait()
    pltpu.make_async_copy(query_h.at[pl.ds(0, CH)], rows_q.at[slot], sem).wait()

  def drain_out(slot, sem):
    pltpu.make_async_copy(out_v.at[slot], out_h.at[pl.ds(0, CH)], sem).wait()

  pltpu.sync_copy(eref_h.at[pl.ds(start * CH, CH)], idx_r.at[0])
  pltpu.sync_copy(equery_h.at[pl.ds(start * CH, CH)], idx_q.at[0])
  transform_idx(0)
  fire(0, sem0)
  stage_idx(1, start + 1)

  def chunk_body(j, carry):
    p = lax.rem(j, 2)
    has_next = j + 1 < count

    @pl.when(jnp.logical_and(has_next, p == 0))
    def _():
      drain_idx(1)
      transform_idx(1)
      fire(1, sem1)

    @pl.when(jnp.logical_and(has_next, p == 1))
    def _():
      drain_idx(0)
      transform_idx(0)
      fire(0, sem0)

    @pl.when(p == 0)
    def _():
      drain(0, sem0)

    @pl.when(p == 1)
    def _():
      drain(1, sem1)

    @pl.when(jnp.logical_and(j >= 2, p == 0))
    def _():
      drain_out(0, semo0)

    @pl.when(jnp.logical_and(j >= 2, p == 1))
    def _():
      drain_out(1, semo1)

    rr = rows_r.at[p]
    rq = rows_q.at[p]
    ov = out_v.at[p]

    @plsc.parallel_loop(0, CH // LANES, 1, unroll=4)
    def _(g):
      sl = pl.ds(g * LANES, LANES)
      pos = g * LANES + iota
      cbr = lax.shift_left(idx_r[p, sl] & 3, 2)
      cbq = lax.shift_left(idx_q[p, sl] & 3, 2)
      r1 = plsc.load_gather(rr, [pos, cbr + 1]) - plsc.load_gather(rq, [pos, cbq + 1])
      r2 = plsc.load_gather(rr, [pos, cbr + 2]) - plsc.load_gather(rq, [pos, cbq + 2])
      r3 = plsc.load_gather(rr, [pos, cbr + 3]) - plsc.load_gather(rq, [pos, cbq + 3])
      o1 = (r1 >= hx).astype(jnp.int32) + (r1 > -hx).astype(jnp.int32)
      o2 = (r2 >= hy).astype(jnp.int32) + (r2 > -hy).astype(jnp.int32)
      o3 = (r3 >= hz).astype(jnp.int32) + (r3 > -hz).astype(jnp.int32)
      ov[sl] = (o3 * 3 + o2) * 3 + o1

    @pl.when(j + 2 < count)
    def _():
      stage_idx(p, start + j + 2)

    @pl.when(p == 0)
    def _():
      pltpu.async_copy(out_v.at[0], out_h.at[pl.ds((start + j) * CH, CH)], semo0)

    @pl.when(p == 1)
    def _():
      pltpu.async_copy(out_v.at[1], out_h.at[pl.ds((start + j) * CH, CH)], semo1)

    return carry

  lax.fori_loop(0, count, chunk_body, 0)
  drain_out(0, semo0)
  drain_out(1, semo1)


def kernel(ref, query, e_ref, e_query, half_voxel_size):
  e = e_ref.shape[0]
  assert e % CH == 0, e
  assert (e // CH) // NW >= 2, e
  n, w = ref.shape
  assert w == 4 and ref.shape == query.shape and n % 4 == 0
  ref16 = ref.reshape(n // 4, ROWW)
  query16 = query.reshape(n // 4, ROWW)
  hvec = jnp.broadcast_to(half_voxel_size.astype(jnp.float32).reshape(3, 1),
                          (3, LANES))

  mesh = plsc.VectorSubcoreMesh(core_axis_name="c", subcore_axis_name="s")
  run = pl.kernel(
      _body,
      out_type=jax.ShapeDtypeStruct((e,), jnp.int32),
      mesh=mesh,
      scratch_types=[
          pltpu.VMEM((2, CH), jnp.int32),
          pltpu.VMEM((2, CH), jnp.int32),
          pltpu.VMEM((2, CH), jnp.int32),
          pltpu.VMEM((2, CH), jnp.int32),
          pltpu.VMEM((2, CH, ROWW), jnp.float32),
          pltpu.VMEM((2, CH, ROWW), jnp.float32),
          pltpu.VMEM((2, CH), jnp.int32),
          pltpu.VMEM((3, LANES), jnp.float32),
          pltpu.SemaphoreType.DMA,
          pltpu.SemaphoreType.DMA,
          pltpu.SemaphoreType.DMA,
          pltpu.SemaphoreType.DMA,
          pltpu.SemaphoreType.DMA,
      ],
      compiler_params=pltpu.CompilerParams(
          needs_layout_passes=False, use_tc_tiling_on_sc=False),
  )
  return run(ref16, query16, e_ref, e_query, hvec)


# speedup vs baseline: 32.8402x; 1.0092x over previous
import jax
import jax.numpy as jnp
from jax import lax
from jax.experimental import pallas as pl
from jax.experimental.pallas import tpu as pltpu
from jax.experimental.pallas import tpu_sc as plsc

NC = 2
NS = 16
NW = NC * NS
LANES = 16
ROWW = 16
BLK = 128
K = 5
CH = K * BLK
NSLOT = 3


def _body(ref_h, query_h, eref_h, equery_h, hvec_h, out_h,
          idx_r, idx_q, idxh_r, idxh_q, rows_r, rows_q, out_v, hv_v,
          sem0, sem1, sem2, semi, semo0, semo1):
  n_chunks = out_h.shape[0] // CH
  gsems = (sem0, sem1, sem2)

  wid = lax.axis_index("s") * NC + lax.axis_index("c")
  base_cnt = n_chunks // NW
  rem = n_chunks % NW
  count = base_cnt + jnp.where(wid < rem, 1, 0)
  start = wid * base_cnt + jnp.minimum(wid, rem)

  pltpu.sync_copy(hvec_h, hv_v)
  hx = hv_v[0]
  hy = hv_v[1]
  hz = hv_v[2]

  iota = lax.broadcasted_iota(jnp.int32, (LANES,), 0)

  def stage_idx(slot, c):
    pltpu.async_copy(eref_h.at[pl.ds(c * CH, CH)], idx_r.at[slot], semi)
    pltpu.async_copy(equery_h.at[pl.ds(c * CH, CH)], idx_q.at[slot], semi)

  def drain_idx(slot):
    pltpu.make_async_copy(eref_h.at[pl.ds(0, CH)], idx_r.at[slot], semi).wait()
    pltpu.make_async_copy(equery_h.at[pl.ds(0, CH)], idx_q.at[slot], semi).wait()

  def transform_idx(slot):
    @plsc.parallel_loop(0, CH // LANES, 1, unroll=8)
    def _(g):
      sl = pl.ds(g * LANES, LANES)
      idxh_r[slot, sl] = lax.shift_right_logical(idx_r[slot, sl], 2)
      idxh_q[slot, sl] = lax.shift_right_logical(idx_q[slot, sl], 2)

  def fire(slot, sem):
    for k in range(K):
      pltpu.async_copy(ref_h.at[idxh_r.at[slot, pl.ds(k * BLK, BLK)]],
                       rows_r.at[slot, pl.ds(k * BLK, BLK)], sem)
      pltpu.async_copy(query_h.at[idxh_q.at[slot, pl.ds(k * BLK, BLK)]],
                       rows_q.at[slot, pl.ds(k * BLK, BLK)], sem)

  def drain(slot, sem):
    pltpu.make_async_copy(ref_h.at[pl.ds(0, CH)], rows_r.at[slot], sem).wait()
    pltpu.make_async_copy(query_h.at[pl.ds(0, CH)], rows_q.at[slot], sem).wait()

  def drain_out(slot, sem):
    pltpu.make_async_copy(out_v.at[slot], out_h.at[pl.ds(0, CH)], sem).wait()

  def ready(slot):
    drain_idx(slot)
    transform_idx(slot)
    fire(slot, gsems[slot])

  pltpu.sync_copy(eref_h.at[pl.ds(start * CH, CH)], idx_r.at[0])
  pltpu.sync_copy(equery_h.at[pl.ds(start * CH, CH)], idx_q.at[0])
  pltpu.sync_copy(eref_h.at[pl.ds((start + 1) * CH, CH)], idx_r.at[1])
  pltpu.sync_copy(equery_h.at[pl.ds((start + 1) * CH, CH)], idx_q.at[1])
  transform_idx(0)
  fire(0, sem0)
  transform_idx(1)
  fire(1, sem1)
  stage_idx(2, start + 2)

  def chunk_body(j, carry):
    p = lax.rem(j, NSLOT)
    po = lax.rem(j, 2)
    pn2 = lax.rem(j + 2, NSLOT)

    for s in range(NSLOT):
      @pl.when(jnp.logical_and(j + 2 < count, pn2 == s))
      def _():
        ready(s)

    for s in range(NSLOT):
      @pl.when(p == s)
      def _():
        drain(s, gsems[s])

    @pl.when(jnp.logical_and(j >= 2, po == 0))
    def _():
      drain_out(0, semo0)

    @pl.when(jnp.logical_and(j >= 2, po == 1))
    def _():
      drain_out(1, semo1)

    rr = rows_r.at[p]
    rq = rows_q.at[p]
    ov = out_v.at[po]

    @plsc.parallel_loop(0, CH // LANES, 1, unroll=4)
    def _(g):
      sl = pl.ds(g * LANES, LANES)
      pos = g * LANES + iota
      cbr = lax.shift_left(idx_r[p, sl] & 3, 2)
      cbq = lax.shift_left(idx_q[p, sl] & 3, 2)
      r1 = plsc.load_gather(rr, [pos, cbr + 1]) - plsc.load_gather(rq, [pos, cbq + 1])
      r2 = plsc.load_gather(rr, [pos, cbr + 2]) - plsc.load_gather(rq, [pos, cbq + 2])
      r3 = plsc.load_gather(rr, [pos, cbr + 3]) - plsc.load_gather(rq, [pos, cbq + 3])
      o1 = (r1 >= hx).astype(jnp.int32) + (r1 > -hx).astype(jnp.int32)
      o2 = (r2 >= hy).astype(jnp.int32) + (r2 > -hy).astype(jnp.int32)
      o3 = (r3 >= hz).astype(jnp.int32) + (r3 > -hz).astype(jnp.int32)
      ov[sl] = (o3 * 3 + o2) * 3 + o1

    @pl.when(j + 3 < count)
    def _():
      stage_idx(p, start + j + 3)

    @pl.when(po == 0)
    def _():
      pltpu.async_copy(out_v.at[0], out_h.at[pl.ds((start + j) * CH, CH)], semo0)

    @pl.when(po == 1)
    def _():
      pltpu.async_copy(out_v.at[1], out_h.at[pl.ds((start + j) * CH, CH)], semo1)

    return carry

  lax.fori_loop(0, count, chunk_body, 0)
  drain_out(0, semo0)
  drain_out(1, semo1)


def kernel(ref, query, e_ref, e_query, half_voxel_size):
  e = e_ref.shape[0]
  assert e % CH == 0, e
  assert (e // CH) // NW >= 3, e
  n, w = ref.shape
  assert w == 4 and ref.shape == query.shape and n % 4 == 0
  ref16 = ref.reshape(n // 4, ROWW)
  query16 = query.reshape(n // 4, ROWW)
  hvec = jnp.broadcast_to(half_voxel_size.astype(jnp.float32).reshape(3, 1),
                          (3, LANES))

  mesh = plsc.VectorSubcoreMesh(core_axis_name="c", subcore_axis_name="s")
  run = pl.kernel(
      _body,
      out_type=jax.ShapeDtypeStruct((e,), jnp.int32),
      mesh=mesh,
      scratch_types=[
          pltpu.VMEM((NSLOT, CH), jnp.int32),
          pltpu.VMEM((NSLOT, CH), jnp.int32),
          pltpu.VMEM((NSLOT, CH), jnp.int32),
          pltpu.VMEM((NSLOT, CH), jnp.int32),
          pltpu.VMEM((NSLOT, CH, ROWW), jnp.float32),
          pltpu.VMEM((NSLOT, CH, ROWW), jnp.float32),
          pltpu.VMEM((2, CH), jnp.int32),
          pltpu.VMEM((3, LANES), jnp.float32),
          pltpu.SemaphoreType.DMA,
          pltpu.SemaphoreType.DMA,
          pltpu.SemaphoreType.DMA,
          pltpu.SemaphoreType.DMA,
          pltpu.SemaphoreType.DMA,
          pltpu.SemaphoreType.DMA,
      ],
      compiler_params=pltpu.CompilerParams(
          needs_layout_passes=False, use_tc_tiling_on_sc=False),
  )
  return run(ref16, query16, e_ref, e_query, hvec)
